# baseline (speedup 1.0000x reference)
import functools

import jax
import jax.numpy as jnp
from jax import lax
from jax.experimental import pallas as pl
from jax.experimental.pallas import tpu as pltpu
from jax.experimental.pallas import tpu_sc as plsc

N = 160000
D = 256
H = 512
O = 256
S = 1024

PADR = 512
NP = N + PADR


BLK = 1600
GRID = N // BLK


def _mlp_body(x_ref, w1_ref, b1_ref, w2_ref, b2_ref, o_ref):
    h = jnp.tanh(
        jnp.dot(x_ref[...], w1_ref[...], preferred_element_type=jnp.float32)
        + b1_ref[...]
    )
    o_ref[...] = jnp.tanh(
        jnp.dot(h, w2_ref[...], preferred_element_type=jnp.float32) + b2_ref[...]
    )


def _mlp(x, w1, b1, w2, b2):
    return pl.pallas_call(
        _mlp_body,
        grid=(GRID,),
        in_specs=[
            pl.BlockSpec((BLK, D), lambda i: (i, 0)),
            pl.BlockSpec((D, H), lambda i: (0, 0)),
            pl.BlockSpec((1, H), lambda i: (0, 0)),
            pl.BlockSpec((H, O), lambda i: (0, 0)),
            pl.BlockSpec((1, O), lambda i: (0, 0)),
        ],
        out_specs=pl.BlockSpec((BLK, O), lambda i: (i, 0)),
        out_shape=jax.ShapeDtypeStruct((NP, O), jnp.float32),
    )(x, w1, b1.reshape(1, H), w2, b2.reshape(1, O))


NC = 2
NS = 16
NW = NC * NS
SPT = S // NW
CH = 224
CBUF = CH + 8
NV = O // 16


def _seg_body(y_hbm, bnd_hbm, out_hbm, ybufs, bndbuf, acc, ysems):
    cid = lax.axis_index("c")
    sid = lax.axis_index("s")
    wid = cid * NS + sid
    seg0 = wid * SPT

    pltpu.sync_copy(bnd_hbm.at[pl.ds(seg0, 48)], bndbuf)
    bv = [bndbuf[pl.ds(0, 16)], bndbuf[pl.ds(16, 16)], bndbuf[pl.ds(32, 16)]]
    sv = [bv[s // 16][s % 16] for s in range(SPT + 1)]
    lo = sv[0]
    hi = sv[SPT]

    @pl.loop(0, SPT * NV)
    def _zr(r):
        acc[pl.ds(r * 16, 16)] = jnp.zeros((16,), jnp.float32)

    zvec = jnp.zeros((16,), jnp.float32)
    npairs = jnp.maximum(1, (hi - lo + (2 * CH - 1)) // (2 * CH))
    nchunks = 2 * npairs

    def chunk_start(c, b):
        start = lo + c * CH
        cs = (start // 8) * 8
        pltpu.async_copy(y_hbm.at[pl.ds(cs, CBUF)], ybufs[b], ysems[b])

    def chunk_wait(b):
        pltpu.make_async_copy(y_hbm.at[pl.ds(0, CBUF)], ybufs[b], ysems[b]).wait()

    def process(c, b):
        start = lo + c * CH
        cs = (start // 8) * 8
        ybuf = ybufs[b]
        cend = start + CH

        for s in range(SPT):
            lo_s = jnp.maximum(sv[s], start)
            hi_s = jnp.minimum(sv[s + 1], cend)

            for half in range(2):
                hbase = half * (NV // 2) * 16

                def row_body(r, a, hbase=hbase):
                    rb = r - cs
                    return tuple(
                        a[t] + ybuf[rb, pl.ds(hbase + t * 16, 16)]
                        for t in range(NV // 2)
                    )

                a = lax.fori_loop(
                    lo_s, hi_s, row_body, tuple(zvec for _ in range(NV // 2))
                )

                @pl.when(hi_s > lo_s)
                def _(a=a, hbase=hbase):
                    for t in range(NV // 2):
                        acc[pl.ds(s * O + hbase + t * 16, 16)] = (
                            acc[pl.ds(s * O + hbase + t * 16, 16)] + a[t]
                        )

    chunk_start(0, 0)

    def pair_body(g, carry):
        for b in range(2):
            c = 2 * g + b
            chunk_wait(b)

            @pl.when(c + 1 < nchunks)
            def _():
                chunk_start(c + 1, 1 - b)

            process(c, b)
        return carry

    lax.fori_loop(0, npairs, pair_body, jnp.int32(0))

    pltpu.sync_copy(acc, out_hbm.at[pl.ds(seg0 * O, SPT * O)])


@functools.partial(
    pl.kernel,
    out_type=jax.ShapeDtypeStruct((S * O,), jnp.float32),
    mesh=plsc.VectorSubcoreMesh(core_axis_name="c", subcore_axis_name="s"),
    scratch_types=[
        pltpu.VMEM((CBUF, O), jnp.float32),
        pltpu.VMEM((CBUF, O), jnp.float32),
        pltpu.VMEM((48,), jnp.int32),
        pltpu.VMEM((SPT * O,), jnp.float32),
        pltpu.SemaphoreType.DMA,
        pltpu.SemaphoreType.DMA,
    ],
)
def _segsum(y_hbm, bnd_hbm, out_hbm, ybuf0, ybuf1, bndbuf, acc, ys0, ys1):
    _seg_body(y_hbm, bnd_hbm, out_hbm, (ybuf0, ybuf1), bndbuf, acc, (ys0, ys1))


def kernel(atom_features, atom_membership, W1, b1, W2, b2):
    y = _mlp(atom_features, W1, b1, W2, b2)
    edges = jnp.arange(0, S + 1, dtype=jnp.int32)
    starts = jnp.searchsorted(atom_membership, edges, side="left").astype(jnp.int32)
    starts = jnp.pad(starts, (0, 1040 - (S + 1)), constant_values=N)
    return _segsum(y, starts).reshape(S, O)


# speedup vs baseline: 2.1006x; 2.1006x over previous
import functools

import jax
import jax.numpy as jnp
from jax import lax
from jax.experimental import pallas as pl
from jax.experimental.pallas import tpu as pltpu
from jax.experimental.pallas import tpu_sc as plsc

N = 160000
D = 256
H = 512
O = 256
S = 1024

PADR = 512
NP = N + PADR


BLK = 1600
GRID = N // BLK


def _mlp_body(x_ref, w1_ref, b1_ref, w2_ref, b2_ref, o_ref):
    h = jnp.tanh(
        jnp.dot(x_ref[...], w1_ref[...], preferred_element_type=jnp.float32)
        + b1_ref[...]
    )
    o_ref[...] = jnp.tanh(
        jnp.dot(h, w2_ref[...], preferred_element_type=jnp.float32) + b2_ref[...]
    )


def _mlp(x, w1, b1, w2, b2):
    return pl.pallas_call(
        _mlp_body,
        grid=(GRID,),
        in_specs=[
            pl.BlockSpec((BLK, D), lambda i: (i, 0)),
            pl.BlockSpec((D, H), lambda i: (0, 0)),
            pl.BlockSpec((1, H), lambda i: (0, 0)),
            pl.BlockSpec((H, O), lambda i: (0, 0)),
            pl.BlockSpec((1, O), lambda i: (0, 0)),
        ],
        out_specs=pl.BlockSpec((BLK, O), lambda i: (i, 0)),
        out_shape=jax.ShapeDtypeStruct((NP, O), jnp.float32),
    )(x, w1, b1.reshape(1, H), w2, b2.reshape(1, O))


NC = 2
NS = 16
NW = NC * NS
SPT = S // NW
CH = 216
CBUF = CH + 8
NV = O // 16
MCH = 5120


def _seg_body(y_hbm, mem_hbm, bnd_hbm, out_hbm, ybufs, mbuf, bndbuf, acc, ysems):
    cid = lax.axis_index("c")
    sid = lax.axis_index("s")
    wid = cid * NS + sid
    seg0 = wid * SPT

    pltpu.sync_copy(bnd_hbm, bndbuf)
    bvec = bndbuf[pl.ds(wid, 16)]
    lo = bvec[0]
    hi = bvec[1]

    cs0 = (lo // 8) * 8
    nmch = jnp.maximum(1, (hi - cs0 + (MCH - 1)) // MCH)

    def mchunk(q, cnts):
        cbeg = cs0 + q * MCH
        pltpu.sync_copy(mem_hbm.at[pl.ds(cbeg, MCH)], mbuf.at[pl.ds(0, MCH)])
        wlo = jnp.clip(lo - cbeg, 0, MCH)
        whi = jnp.clip(hi - cbeg, 0, MCH)
        new = []
        for e in range(1, SPT):
            edge = seg0 + e
            pos = jnp.int32(0)
            st = MCH // 2
            while st >= 1:
                cand = pos + st
                v = mbuf[pl.ds(cand - 1, 16)][0]
                pos = jnp.where(
                    jnp.logical_and(cand <= MCH, v < edge), cand, pos
                )
                st //= 2
            new.append(cnts[e - 1] + jnp.clip(pos, wlo, whi) - wlo)
        return tuple(new)

    cnts = lax.fori_loop(
        0, nmch, mchunk, tuple(jnp.int32(0) for _ in range(SPT - 1))
    )
    sv = [lo] + [lo + cnts[e - 1] for e in range(1, SPT)] + [hi]

    @pl.loop(0, SPT * NV)
    def _zr(r):
        acc[pl.ds(r * 16, 16)] = jnp.zeros((16,), jnp.float32)

    zvec = jnp.zeros((16,), jnp.float32)
    npairs = jnp.maximum(1, (hi - lo + (2 * CH - 1)) // (2 * CH))
    nchunks = 2 * npairs

    def chunk_start(c, b):
        start = lo + c * CH
        cs = (start // 8) * 8
        pltpu.async_copy(y_hbm.at[pl.ds(cs, CBUF)], ybufs[b], ysems[b])

    def chunk_wait(b):
        pltpu.make_async_copy(y_hbm.at[pl.ds(0, CBUF)], ybufs[b], ysems[b]).wait()

    def process(c, b):
        start = lo + c * CH
        cs = (start // 8) * 8
        ybuf = ybufs[b]
        cend = start + CH

        for s in range(SPT):
            lo_s = jnp.maximum(sv[s], start)
            hi_s = jnp.minimum(sv[s + 1], cend)

            for half in range(2):
                hbase = half * (NV // 2) * 16

                def row_body(r, a, hbase=hbase):
                    rb = r - cs
                    return tuple(
                        a[t] + ybuf[rb, pl.ds(hbase + t * 16, 16)]
                        for t in range(NV // 2)
                    )

                a = lax.fori_loop(
                    lo_s, hi_s, row_body, tuple(zvec for _ in range(NV // 2))
                )

                @pl.when(hi_s > lo_s)
                def _(a=a, hbase=hbase):
                    for t in range(NV // 2):
                        acc[pl.ds(s * O + hbase + t * 16, 16)] = (
                            acc[pl.ds(s * O + hbase + t * 16, 16)] + a[t]
                        )

    chunk_start(0, 0)

    def pair_body(g, carry):
        for b in range(2):
            c = 2 * g + b
            chunk_wait(b)

            @pl.when(c + 1 < nchunks)
            def _():
                chunk_start(c + 1, 1 - b)

            process(c, b)
        return carry

    lax.fori_loop(0, npairs, pair_body, jnp.int32(0))

    pltpu.sync_copy(acc, out_hbm.at[pl.ds(seg0 * O, SPT * O)])


@functools.partial(
    pl.kernel,
    out_type=jax.ShapeDtypeStruct((S * O,), jnp.float32),
    mesh=plsc.VectorSubcoreMesh(core_axis_name="c", subcore_axis_name="s"),
    scratch_types=[
        pltpu.VMEM((CBUF, O), jnp.float32),
        pltpu.VMEM((CBUF, O), jnp.float32),
        pltpu.VMEM((MCH + 16,), jnp.int32),
        pltpu.VMEM((48,), jnp.int32),
        pltpu.VMEM((SPT * O,), jnp.float32),
        pltpu.SemaphoreType.DMA,
        pltpu.SemaphoreType.DMA,
    ],
)
def _segsum(y_hbm, mem_hbm, bnd_hbm, out_hbm,
            ybuf0, ybuf1, mbuf, bndbuf, acc, ys0, ys1):
    _seg_body(y_hbm, mem_hbm, bnd_hbm, out_hbm,
              (ybuf0, ybuf1), mbuf, bndbuf, acc, (ys0, ys1))


def kernel(atom_features, atom_membership, W1, b1, W2, b2):
    y = _mlp(atom_features, W1, b1, W2, b2)
    edges = jnp.arange(0, S + 1, SPT, dtype=jnp.int32)
    bounds = jnp.searchsorted(atom_membership, edges, side="left").astype(jnp.int32)
    bounds = jnp.pad(bounds, (0, 48 - (NW + 1)))
    mem_pad = jnp.pad(atom_membership, (0, MCH + 16), constant_values=S)
    return _segsum(y, mem_pad, bounds).reshape(S, O)


# speedup vs baseline: 2.1080x; 1.0035x over previous
import functools

import jax
import jax.numpy as jnp
from jax import lax
from jax.experimental import pallas as pl
from jax.experimental.pallas import tpu as pltpu
from jax.experimental.pallas import tpu_sc as plsc

N = 160000
D = 256
H = 512
O = 256
S = 1024

PADR = 512
NP = N + PADR


BLK = 1600
GRID = N // BLK


def _mlp_body(x_ref, w1_ref, b1_ref, w2_ref, b2_ref, o_ref):
    h = jnp.tanh(
        jnp.dot(x_ref[...], w1_ref[...], preferred_element_type=jnp.float32)
        + b1_ref[...]
    )
    o_ref[...] = jnp.tanh(
        jnp.dot(h, w2_ref[...], preferred_element_type=jnp.float32) + b2_ref[...]
    )


def _mlp(x, w1, b1, w2, b2):
    return pl.pallas_call(
        _mlp_body,
        grid=(GRID,),
        in_specs=[
            pl.BlockSpec((BLK, D), lambda i: (i, 0)),
            pl.BlockSpec((D, H), lambda i: (0, 0)),
            pl.BlockSpec((1, H), lambda i: (0, 0)),
            pl.BlockSpec((H, O), lambda i: (0, 0)),
            pl.BlockSpec((1, O), lambda i: (0, 0)),
        ],
        out_specs=pl.BlockSpec((BLK, O), lambda i: (i, 0)),
        out_shape=jax.ShapeDtypeStruct((NP, O), jnp.float32),
    )(x, w1, b1.reshape(1, H), w2, b2.reshape(1, O))


NC = 2
NS = 16
NW = NC * NS
SPT = S // NW
CH = 216
CBUF = CH + 8
NV = O // 16
MCH = 5120


def _seg_body(y_hbm, mem_hbm, bnd_hbm, out_hbm, ybufs, mbuf, bndbuf, acc, ysems):
    cid = lax.axis_index("c")
    sid = lax.axis_index("s")
    wid = cid * NS + sid
    seg0 = wid * SPT

    pltpu.sync_copy(bnd_hbm, bndbuf)
    bvec = bndbuf[pl.ds(wid, 16)]
    lo = bvec[0]
    hi = bvec[1]

    cs0 = (lo // 8) * 8
    nmch = jnp.maximum(1, (hi - cs0 + (MCH - 1)) // MCH)

    def mchunk(q, cnts):
        cbeg = cs0 + q * MCH
        pltpu.sync_copy(mem_hbm.at[pl.ds(cbeg, MCH)], mbuf.at[pl.ds(0, MCH)])
        wlo = jnp.clip(lo - cbeg, 0, MCH)
        whi = jnp.clip(hi - cbeg, 0, MCH)
        new = []
        for e in range(1, SPT):
            edge = seg0 + e
            pos = jnp.int32(0)
            st = 4096
            while st >= 1:
                cand = pos + st
                v = mbuf[pl.ds(cand - 1, 16)][0]
                pos = jnp.where(
                    jnp.logical_and(cand <= MCH, v < edge), cand, pos
                )
                st //= 2
            new.append(cnts[e - 1] + jnp.clip(pos, wlo, whi) - wlo)
        return tuple(new)

    cnts = lax.fori_loop(
        0, nmch, mchunk, tuple(jnp.int32(0) for _ in range(SPT - 1))
    )
    sv = [lo] + [lo + cnts[e - 1] for e in range(1, SPT)] + [hi]

    @pl.loop(0, SPT * NV)
    def _zr(r):
        acc[pl.ds(r * 16, 16)] = jnp.zeros((16,), jnp.float32)

    zvec = jnp.zeros((16,), jnp.float32)
    npairs = jnp.maximum(1, (hi - lo + (2 * CH - 1)) // (2 * CH))
    nchunks = 2 * npairs

    def chunk_start(c, b):
        start = lo + c * CH
        cs = (start // 8) * 8
        pltpu.async_copy(y_hbm.at[pl.ds(cs, CBUF)], ybufs[b], ysems[b])

    def chunk_wait(b):
        pltpu.make_async_copy(y_hbm.at[pl.ds(0, CBUF)], ybufs[b], ysems[b]).wait()

    def process(c, b):
        start = lo + c * CH
        cs = (start // 8) * 8
        ybuf = ybufs[b]
        cend = start + CH

        for s in range(SPT):
            lo_s = jnp.maximum(sv[s], start)
            hi_s = jnp.minimum(sv[s + 1], cend)

            for half in range(2):
                hbase = half * (NV // 2) * 16

                def row_body(r, a, hbase=hbase):
                    rb = r - cs
                    return tuple(
                        a[t] + ybuf[rb, pl.ds(hbase + t * 16, 16)]
                        for t in range(NV // 2)
                    )

                a = lax.fori_loop(
                    lo_s, hi_s, row_body, tuple(zvec for _ in range(NV // 2))
                )

                @pl.when(hi_s > lo_s)
                def _(a=a, hbase=hbase):
                    for t in range(NV // 2):
                        acc[pl.ds(s * O + hbase + t * 16, 16)] = (
                            acc[pl.ds(s * O + hbase + t * 16, 16)] + a[t]
                        )

    chunk_start(0, 0)

    def pair_body(g, carry):
        for b in range(2):
            c = 2 * g + b
            chunk_wait(b)

            @pl.when(c + 1 < nchunks)
            def _():
                chunk_start(c + 1, 1 - b)

            process(c, b)
        return carry

    lax.fori_loop(0, npairs, pair_body, jnp.int32(0))

    pltpu.sync_copy(acc, out_hbm.at[pl.ds(seg0 * O, SPT * O)])


@functools.partial(
    pl.kernel,
    out_type=jax.ShapeDtypeStruct((S * O,), jnp.float32),
    mesh=plsc.VectorSubcoreMesh(core_axis_name="c", subcore_axis_name="s"),
    scratch_types=[
        pltpu.VMEM((CBUF, O), jnp.float32),
        pltpu.VMEM((CBUF, O), jnp.float32),
        pltpu.VMEM((MCH + 16,), jnp.int32),
        pltpu.VMEM((48,), jnp.int32),
        pltpu.VMEM((SPT * O,), jnp.float32),
        pltpu.SemaphoreType.DMA,
        pltpu.SemaphoreType.DMA,
    ],
)
def _segsum(y_hbm, mem_hbm, bnd_hbm, out_hbm,
            ybuf0, ybuf1, mbuf, bndbuf, acc, ys0, ys1):
    _seg_body(y_hbm, mem_hbm, bnd_hbm, out_hbm,
              (ybuf0, ybuf1), mbuf, bndbuf, acc, (ys0, ys1))


def kernel(atom_features, atom_membership, W1, b1, W2, b2):
    y = _mlp(atom_features, W1, b1, W2, b2)
    edges = jnp.arange(0, S + 1, SPT, dtype=jnp.int32)
    bounds = jnp.searchsorted(atom_membership, edges, side="left").astype(jnp.int32)
    bounds = jnp.pad(bounds, (0, 48 - (NW + 1)))
    mem_pad = jnp.pad(atom_membership, (0, MCH + 16), constant_values=S)
    return _segsum(y, mem_pad, bounds).reshape(S, O)
